# Initial kernel scaffold; baseline (speedup 1.0000x reference)
import jax
import jax.numpy as jnp
from jax.experimental import pallas as pl


def kernel(x, edge_index, edge_attr, positions, batch, params):
    raise NotImplementedError("write your pallas kernel here")


# speedup vs baseline: 2.4936x; 2.4936x over previous
import functools

import jax
import jax.numpy as jnp
from jax import lax
from jax.experimental import pallas as pl
from jax.experimental.pallas import tpu as pltpu, tpu_sc as plsc

N_NODES = 50000
N_EDGES = 800000
D_EDGE = 4
POS_DIM = 2
NUM_GRAPHS = 64

NC, NS, L = 2, 16, 16
NW = NC * NS
CHUNK = 128
EP = 819200
JW = EP // (NW * CHUNK)
NPAD = 53248
ZCH = NPAD // (NS * CHUNK)
IB = 25

BE = 3200
BN = 3328
BP = 2000


def _silu(v):
    return v * jax.nn.sigmoid(v)


def _make_sc_gather(dt):
    mesh = plsc.VectorSubcoreMesh(core_axis_name="c", subcore_axis_name="s",
                                  num_cores=NC, num_subcores=NS)

    @functools.partial(
        pl.kernel, mesh=mesh,
        out_type=(jax.ShapeDtypeStruct((EP, dt), jnp.float32),
                  jax.ShapeDtypeStruct((EP, dt), jnp.float32)),
        scratch_types=[pltpu.VMEM((JW, CHUNK), jnp.int32),
                       pltpu.VMEM((JW, CHUNK), jnp.int32),
                       pltpu.VMEM((CHUNK, dt), jnp.float32),
                       pltpu.VMEM((CHUNK, dt), jnp.float32),
                       pltpu.SemaphoreType.DMA,
                       pltpu.SemaphoreType.DMA],
        compiler_params=pltpu.CompilerParams(use_tc_tiling_on_sc=False),
    )
    def k(tab_hbm, src_hbm, dst_hbm, gs_hbm, gd_hbm,
          idxs_v, idxd_v, rows_v, rowd_v, sem_s, sem_d):
        c = lax.axis_index("c")
        s = lax.axis_index("s")
        w = c * NS + s
        pltpu.sync_copy(src_hbm.at[c, s], idxs_v)
        pltpu.sync_copy(dst_hbm.at[c, s], idxd_v)
        base = w * (JW * CHUNK)

        def body(j, carry):
            cp_s = pltpu.async_copy(tab_hbm.at[idxs_v.at[j]], rows_v, sem_s)
            cp_d = pltpu.async_copy(tab_hbm.at[idxd_v.at[j]], rowd_v, sem_d)
            cp_s.wait()
            pltpu.sync_copy(rows_v, gs_hbm.at[pl.ds(base + j * CHUNK, CHUNK)])
            cp_d.wait()
            pltpu.sync_copy(rowd_v, gd_hbm.at[pl.ds(base + j * CHUNK, CHUNK)])
            return carry

        lax.fori_loop(0, JW, body, 0)

    return k


def _make_sc_scatter(dv):
    mesh = plsc.VectorSubcoreMesh(core_axis_name="c", subcore_axis_name="s",
                                  num_cores=NC, num_subcores=NS)

    @functools.partial(
        pl.kernel, mesh=mesh,
        out_type=jax.ShapeDtypeStruct((NC, NPAD, dv), jnp.float32),
        scratch_types=[pltpu.VMEM_SHARED((NPAD, dv), jnp.float32),
                       pltpu.VMEM((IB, CHUNK), jnp.int32),
                       pltpu.VMEM((CHUNK, dv), jnp.float32)],
        compiler_params=pltpu.CompilerParams(use_tc_tiling_on_sc=False),
    )
    def k(val_hbm, dst_hbm, out_hbm, acc_sh, idx_v, val_v):
        c = lax.axis_index("c")
        s = lax.axis_index("s")

        def zrow(r, carry):
            def zcol(kk, carry2):
                val_v[r, pl.ds(kk * L, L)] = jnp.zeros((L,), jnp.float32)
                return carry2
            return lax.fori_loop(0, dv // L, zcol, carry)
        lax.fori_loop(0, CHUNK, zrow, 0)

        def zacc(t, carry):
            pltpu.sync_copy(val_v, acc_sh.at[pl.ds((s * ZCH + t) * CHUNK, CHUNK)])
            return carry
        lax.fori_loop(0, ZCH, zacc, 0)
        plsc.subcore_barrier()

        base = (c * NS + s) * (JW * CHUNK)

        def blk(b, carry):
            pltpu.sync_copy(dst_hbm.at[c, s, pl.ds(b * IB, IB)], idx_v)

            def body(t, carry2):
                j = b * IB + t
                pltpu.sync_copy(val_hbm.at[pl.ds(base + j * CHUNK, CHUNK)],
                                val_v)
                pltpu.sync_copy(val_v, acc_sh.at[idx_v.at[t]], add=True)
                return carry2
            return lax.fori_loop(0, IB, body, carry)
        lax.fori_loop(0, JW // IB, blk, 0)
        plsc.subcore_barrier()

        def dump(t, carry):
            r0 = (s * ZCH + t) * CHUNK
            pltpu.sync_copy(acc_sh.at[pl.ds(r0, CHUNK)],
                            out_hbm.at[c, pl.ds(r0, CHUNK)])
            return carry
        lax.fori_loop(0, ZCH, dump, 0)

    return k


def _edge_block(gs, gd, ea, f, p, with_coors, pid, mout, cout):
    xj = gs[:, 2:2 + f]
    xi = gd[:, 2:2 + f]
    rel = gs[:, 0:2] - gd[:, 0:2]
    rd = jnp.sum(rel * rel, axis=1, keepdims=True)
    h = (xi @ p['w1i'] + xj @ p['w1j'] + ea @ p['w1e'] + rd * p['w1d']
         + p['b1'])
    h = _silu(h)
    m = _silu(h @ p['w2'] + p['b2'])
    eid = pid * BE + lax.broadcasted_iota(jnp.int32, (BE, 1), 0)
    mask = eid < N_EDGES
    mout[...] = jnp.where(mask, m, 0.0)
    if with_coors:
        ch = _silu(m @ p['cw1'] + p['cb1'])
        cw = jnp.sum(ch * p['cw2t'], axis=1, keepdims=True) + p['cb2']
        norm = jnp.sqrt(jnp.clip(rd, 1e-16))
        rel_n = rel / jnp.maximum(norm, 1e-8) * p['cscale']
        cwrel = cw * rel_n
        cpad = jnp.concatenate([cwrel, jnp.zeros((BE, 14), jnp.float32)], axis=1)
        cout[...] = jnp.where(mask, cpad, 0.0)


def _tc_edge(gs, gd, ea, wp, f, dt, with_coors):
    wkeys = ['w1i', 'w1j', 'w1e', 'w1d', 'b1', 'w2', 'b2']
    if with_coors:
        wkeys += ['cw1', 'cb1', 'cw2t', 'cb2', 'cscale']
    wvals = [wp[k] for k in wkeys]

    def body(gs_r, gd_r, ea_r, *rest):
        nw = len(wvals)
        wrefs = rest[:nw]
        outs = rest[nw:]
        p = {k: wr[...] for k, wr in zip(wkeys, wrefs)}
        mout = outs[0]
        cout = outs[1] if with_coors else None
        _edge_block(gs_r[...], gd_r[...], ea_r[...], f, p, with_coors,
                    pl.program_id(0), mout, cout)

    grid = EP // BE
    full = lambda a: pl.BlockSpec(a.shape, lambda i: (0,) * a.ndim)
    in_specs = [pl.BlockSpec((BE, dt), lambda i: (i, 0)),
                pl.BlockSpec((BE, dt), lambda i: (i, 0)),
                pl.BlockSpec((BE, D_EDGE), lambda i: (i, 0))]
    in_specs += [full(w) for w in wvals]
    out_shape = [jax.ShapeDtypeStruct((EP, 32), jnp.float32)]
    out_specs = [pl.BlockSpec((BE, 32), lambda i: (i, 0))]
    if with_coors:
        out_shape.append(jax.ShapeDtypeStruct((EP, 16), jnp.float32))
        out_specs.append(pl.BlockSpec((BE, 16), lambda i: (i, 0)))
    res = pl.pallas_call(
        body, grid=(grid,), in_specs=in_specs, out_specs=out_specs,
        out_shape=out_shape,
    )(gs, gd, ea, *wvals)
    return res if with_coors else (res[0], None)


def _tc_node(tab, accm, accc, wp, f, dt_in, last):
    wkeys = ['nw1f', 'nw1m', 'nb1', 'nw2', 'nb2']
    wvals = [wp[k] for k in wkeys]
    dt_out = 32 if last else 48

    def body(tab_r, accm_r, *rest):
        if not last:
            accc_r = rest[0]
            rest = rest[1:]
        wrefs = rest[:len(wvals)]
        out_r = rest[len(wvals)]
        p = {k: wr[...] for k, wr in zip(wkeys, wrefs)}
        feats = tab_r[:, 2:2 + f]
        m_i = accm_r[0] + accm_r[1]
        hid = _silu(feats @ p['nw1f'] + m_i @ p['nw1m'] + p['nb1'])
        hid = hid @ p['nw2'] + p['nb2']
        if last:
            out_r[...] = hid
        else:
            mhat = accc_r[0, :, 0:2] + accc_r[1, :, 0:2]
            coors = tab_r[:, 0:2] + mhat
            out_r[...] = jnp.concatenate(
                [coors, hid, jnp.zeros((BN, 14), jnp.float32)], axis=1)

    full = lambda a: pl.BlockSpec(a.shape, lambda i: (0,) * a.ndim)
    in_specs = [pl.BlockSpec((BN, dt_in), lambda i: (i, 0)),
                pl.BlockSpec((NC, BN, 32), lambda i: (0, i, 0))]
    args = [tab, accm]
    if not last:
        in_specs.append(pl.BlockSpec((NC, BN, 16), lambda i: (0, i, 0)))
        args.append(accc)
    in_specs += [full(w) for w in wvals]
    args += wvals
    return pl.pallas_call(
        body, grid=(NPAD // BN,), in_specs=in_specs,
        out_specs=pl.BlockSpec((BN, dt_out), lambda i: (i, 0)),
        out_shape=jax.ShapeDtypeStruct((NPAD, dt_out), jnp.float32),
    )(*args)


def _tc_pool(feats, batch3, lwt, lb):
    nblk = N_NODES // BP

    def body(f_r, b_r, lwt_r, lb_r, out_r, acc):
        i = pl.program_id(0)

        @pl.when(i == 0)
        def _init():
            acc[...] = jnp.zeros((NUM_GRAPHS, 64), jnp.float32)

        gids = lax.broadcasted_iota(jnp.int32, (NUM_GRAPHS, 1), 0)
        onehot = (gids == b_r[0]).astype(jnp.float32)
        sums = lax.dot_general(onehot, f_r[...],
                               (((1,), (0,)), ((), ())))
        cnts = jnp.sum(onehot, axis=1, keepdims=True)
        acc[:, 0:32] += sums
        acc[:, 32:33] += cnts

        @pl.when(i == nblk - 1)
        def _fin():
            mean = acc[:, 0:32] / jnp.maximum(acc[:, 32:33], 1.0)
            out_r[...] = (jnp.sum(mean * lwt_r[...], axis=1, keepdims=True)
                          + lb_r[...])

    return pl.pallas_call(
        body, grid=(nblk,),
        in_specs=[pl.BlockSpec((BP, 32), lambda i: (i, 0)),
                  pl.BlockSpec((1, 1, BP), lambda i: (i, 0, 0)),
                  pl.BlockSpec((1, 32), lambda i: (0, 0)),
                  pl.BlockSpec((1, 1), lambda i: (0, 0))],
        out_specs=pl.BlockSpec((NUM_GRAPHS, 1), lambda i: (0, 0)),
        out_shape=jax.ShapeDtypeStruct((NUM_GRAPHS, 1), jnp.float32),
        scratch_shapes=[pltpu.VMEM((NUM_GRAPHS, 64), jnp.float32)],
    )(feats, batch3, lwt, lb)


def _prep_layer_weights(p, f):
    e_in = 2 * f + D_EDGE + 1
    w1 = p['e_w1']
    return {
        'w1i': w1[:f], 'w1j': w1[f:2 * f], 'w1e': w1[2 * f:2 * f + D_EDGE],
        'w1d': w1[2 * f + D_EDGE:2 * f + D_EDGE + 1],
        'b1': p['e_b1'][None, :],
        'w2': p['e_w2'], 'b2': p['e_b2'][None, :],
        'cw1': p['c_w1'], 'cb1': p['c_b1'][None, :],
        'cw2t': p['c_w2'].T, 'cb2': p['c_b2'][None, :],
        'cscale': p['coors_scale'][None, :],
        'nw1f': p['n_w1'][:f], 'nw1m': p['n_w1'][f:],
        'nb1': p['n_b1'][None, :], 'nw2': p['n_w2'], 'nb2': p['n_b2'][None, :],
    }


def kernel(x, edge_index, edge_attr, positions, batch, params):
    f32 = jnp.float32
    src = edge_index[0]
    dst = edge_index[1]
    pad_e = EP - N_EDGES
    src_p = jnp.pad(src, (0, pad_e)).reshape(NC, NS, JW, CHUNK)
    dst_p = jnp.pad(dst, (0, pad_e)).reshape(NC, NS, JW, CHUNK)
    ea_p = jnp.pad(edge_attr, ((0, pad_e), (0, 0))).astype(f32)

    gather16 = _make_sc_gather(16)
    gather48 = _make_sc_gather(48)
    scat32 = _make_sc_scatter(32)
    scat16 = _make_sc_scatter(16)

    tab = jnp.concatenate([positions.astype(f32), x.astype(f32),
                           jnp.zeros((N_NODES, 12), f32)], axis=1)
    tab = jnp.pad(tab, ((0, NPAD - N_NODES), (0, 0)))

    fdims = (2, 32, 32)
    tdims = (16, 48, 48)
    for li, name in enumerate(('conv1', 'conv2', 'conv3')):
        f = fdims[li]
        dt = tdims[li]
        wp = _prep_layer_weights(params[name], f)
        last = li == 2
        gather = gather16 if dt == 16 else gather48
        gs, gd = gather(tab, src_p, dst_p)
        m, cvals = _tc_edge(gs, gd, ea_p, wp, f, dt, not last)
        accm = scat32(m, dst_p)
        accc = None if last else scat16(cvals, dst_p)
        tab = _tc_node(tab, accm, accc, wp, f, dt, last)

    batch3 = batch.reshape(N_NODES // BP, 1, BP)
    lwt = params['lin_w'].T
    lb = params['lin_b'][None, :]
    return _tc_pool(tab, batch3, lwt, lb)


# speedup vs baseline: 3.1487x; 1.2627x over previous
import functools

import jax
import jax.numpy as jnp
from jax import lax
from jax.experimental import pallas as pl
from jax.experimental.pallas import tpu as pltpu, tpu_sc as plsc

N_NODES = 50000
N_EDGES = 800000
D_EDGE = 4
POS_DIM = 2
NUM_GRAPHS = 64

NC, NS, L = 2, 16, 16
NW = NC * NS
CHUNK = 128
EP = 819200
JW = EP // (NW * CHUNK)
NPAD = 53248
ZCH = NPAD // (NS * CHUNK)
IB = 50

BE = 3200
BN = 3328
BP = 2000


def _silu(v):
    return v * jax.nn.sigmoid(v)


def _make_sc_gather(dt):
    mesh = plsc.VectorSubcoreMesh(core_axis_name="c", subcore_axis_name="s",
                                  num_cores=NC, num_subcores=NS)

    @functools.partial(
        pl.kernel, mesh=mesh,
        out_type=(jax.ShapeDtypeStruct((EP, dt), jnp.float32),
                  jax.ShapeDtypeStruct((EP, dt), jnp.float32)),
        scratch_types=[pltpu.VMEM((JW, CHUNK), jnp.int32),
                       pltpu.VMEM((JW, CHUNK), jnp.int32),
                       pltpu.VMEM((2, CHUNK, dt), jnp.float32),
                       pltpu.VMEM((2, CHUNK, dt), jnp.float32),
                       pltpu.SemaphoreType.DMA,
                       pltpu.SemaphoreType.DMA],
        compiler_params=pltpu.CompilerParams(use_tc_tiling_on_sc=False),
    )
    def k(tab_hbm, src_hbm, dst_hbm, gs_hbm, gd_hbm,
          idxs_v, idxd_v, rows_v, rowd_v, sem_s, sem_d):
        c = lax.axis_index("c")
        s = lax.axis_index("s")
        w = c * NS + s
        pltpu.sync_copy(src_hbm.at[c, s], idxs_v)
        pltpu.sync_copy(dst_hbm.at[c, s], idxd_v)
        base = w * (JW * CHUNK)

        pltpu.async_copy(tab_hbm.at[idxs_v.at[0]], rows_v.at[0], sem_s)
        pltpu.async_copy(tab_hbm.at[idxd_v.at[0]], rowd_v.at[0], sem_d)

        def body(j, carry):
            nb = (j + 1) % 2
            cb = j % 2

            @pl.when(j + 1 < JW)
            def _fire():
                pltpu.async_copy(tab_hbm.at[idxs_v.at[j + 1]], rows_v.at[nb],
                                 sem_s)
                pltpu.async_copy(tab_hbm.at[idxd_v.at[j + 1]], rowd_v.at[nb],
                                 sem_d)

            pltpu.make_async_copy(tab_hbm.at[idxs_v.at[j]], rows_v.at[cb],
                                  sem_s).wait()
            pltpu.sync_copy(rows_v.at[cb],
                            gs_hbm.at[pl.ds(base + j * CHUNK, CHUNK)])
            pltpu.make_async_copy(tab_hbm.at[idxd_v.at[j]], rowd_v.at[cb],
                                  sem_d).wait()
            pltpu.sync_copy(rowd_v.at[cb],
                            gd_hbm.at[pl.ds(base + j * CHUNK, CHUNK)])
            return carry

        lax.fori_loop(0, JW, body, 0)

    return k


def _make_sc_scatter(dv):
    mesh = plsc.VectorSubcoreMesh(core_axis_name="c", subcore_axis_name="s",
                                  num_cores=NC, num_subcores=NS)

    @functools.partial(
        pl.kernel, mesh=mesh,
        out_type=jax.ShapeDtypeStruct((NC, NPAD, dv), jnp.float32),
        scratch_types=[pltpu.VMEM_SHARED((NPAD, dv), jnp.float32),
                       pltpu.VMEM((IB, CHUNK), jnp.int32),
                       pltpu.VMEM((2, CHUNK, dv), jnp.float32),
                       pltpu.SemaphoreType.DMA],
        compiler_params=pltpu.CompilerParams(use_tc_tiling_on_sc=False),
    )
    def k(val_hbm, dst_hbm, out_hbm, acc_sh, idx_v, val_v, sem_v):
        c = lax.axis_index("c")
        s = lax.axis_index("s")

        def zrow(r, carry):
            def zcol(kk, carry2):
                val_v[0, r, pl.ds(kk * L, L)] = jnp.zeros((L,), jnp.float32)
                return carry2
            return lax.fori_loop(0, dv // L, zcol, carry)
        lax.fori_loop(0, CHUNK, zrow, 0)

        def zacc(t, carry):
            pltpu.sync_copy(val_v.at[0],
                            acc_sh.at[pl.ds((s * ZCH + t) * CHUNK, CHUNK)])
            return carry
        lax.fori_loop(0, ZCH, zacc, 0)
        plsc.subcore_barrier()

        base = (c * NS + s) * (JW * CHUNK)

        def blk(b, carry):
            pltpu.sync_copy(dst_hbm.at[c, s, pl.ds(b * IB, IB)], idx_v)
            j0 = b * IB
            pltpu.async_copy(val_hbm.at[pl.ds(base + j0 * CHUNK, CHUNK)],
                             val_v.at[0], sem_v)

            def body(t, carry2):
                j = j0 + t

                @pl.when(t + 1 < IB)
                def _fire():
                    pltpu.async_copy(
                        val_hbm.at[pl.ds(base + (j + 1) * CHUNK, CHUNK)],
                        val_v.at[(t + 1) % 2], sem_v)

                pltpu.make_async_copy(
                    val_hbm.at[pl.ds(base + j * CHUNK, CHUNK)],
                    val_v.at[t % 2], sem_v).wait()
                pltpu.sync_copy(val_v.at[t % 2], acc_sh.at[idx_v.at[t]],
                                add=True)
                return carry2
            return lax.fori_loop(0, IB, body, carry)
        lax.fori_loop(0, JW // IB, blk, 0)
        plsc.subcore_barrier()

        def dump(t, carry):
            r0 = (s * ZCH + t) * CHUNK
            pltpu.sync_copy(acc_sh.at[pl.ds(r0, CHUNK)],
                            out_hbm.at[c, pl.ds(r0, CHUNK)])
            return carry
        lax.fori_loop(0, ZCH, dump, 0)

    return k


def _edge_block(gs, gd, ea, f, p, with_coors, pid, mout, cout):
    xj = gs[:, 2:2 + f]
    xi = gd[:, 2:2 + f]
    rel = gs[:, 0:2] - gd[:, 0:2]
    rd = jnp.sum(rel * rel, axis=1, keepdims=True)
    ea_term = lax.dot_general(ea, p['w1e'], (((0,), (0,)), ((), ())))
    h = (xi @ p['w1i'] + xj @ p['w1j'] + ea_term + rd * p['w1d']
         + p['b1'])
    h = _silu(h)
    m = _silu(h @ p['w2'] + p['b2'])
    eid = pid * BE + lax.broadcasted_iota(jnp.int32, (BE, 1), 0)
    mask = eid < N_EDGES
    mout[...] = jnp.where(mask, m, 0.0)
    if with_coors:
        ch = _silu(m @ p['cw1'] + p['cb1'])
        cw = jnp.sum(ch * p['cw2t'], axis=1, keepdims=True) + p['cb2']
        norm = jnp.sqrt(jnp.clip(rd, 1e-16))
        rel_n = rel / jnp.maximum(norm, 1e-8) * p['cscale']
        cwrel = cw * rel_n
        cpad = jnp.concatenate([cwrel, jnp.zeros((BE, 14), jnp.float32)], axis=1)
        cout[...] = jnp.where(mask, cpad, 0.0)


def _tc_edge(gs, gd, ea, wp, f, dt, with_coors):
    wkeys = ['w1i', 'w1j', 'w1e', 'w1d', 'b1', 'w2', 'b2']
    if with_coors:
        wkeys += ['cw1', 'cb1', 'cw2t', 'cb2', 'cscale']
    wvals = [wp[k] for k in wkeys]

    def body(gs_r, gd_r, ea_r, *rest):
        nw = len(wvals)
        wrefs = rest[:nw]
        outs = rest[nw:]
        p = {k: wr[...] for k, wr in zip(wkeys, wrefs)}
        mout = outs[0]
        cout = outs[1] if with_coors else None
        _edge_block(gs_r[...], gd_r[...], ea_r[...], f, p, with_coors,
                    pl.program_id(0), mout, cout)

    grid = EP // BE
    full = lambda a: pl.BlockSpec(a.shape, lambda i: (0,) * a.ndim)
    in_specs = [pl.BlockSpec((BE, dt), lambda i: (i, 0)),
                pl.BlockSpec((BE, dt), lambda i: (i, 0)),
                pl.BlockSpec((D_EDGE, BE), lambda i: (0, i))]
    in_specs += [full(w) for w in wvals]
    out_shape = [jax.ShapeDtypeStruct((EP, 32), jnp.float32)]
    out_specs = [pl.BlockSpec((BE, 32), lambda i: (i, 0))]
    if with_coors:
        out_shape.append(jax.ShapeDtypeStruct((EP, 16), jnp.float32))
        out_specs.append(pl.BlockSpec((BE, 16), lambda i: (i, 0)))
    res = pl.pallas_call(
        body, grid=(grid,), in_specs=in_specs, out_specs=out_specs,
        out_shape=out_shape,
    )(gs, gd, ea, *wvals)
    return res if with_coors else (res[0], None)


def _tc_node(tab, accm, accc, wp, f, dt_in, last):
    wkeys = ['nw1f', 'nw1m', 'nb1', 'nw2', 'nb2']
    wvals = [wp[k] for k in wkeys]
    dt_out = 32 if last else 48

    def body(tab_r, accm_r, *rest):
        if not last:
            accc_r = rest[0]
            rest = rest[1:]
        wrefs = rest[:len(wvals)]
        out_r = rest[len(wvals)]
        p = {k: wr[...] for k, wr in zip(wkeys, wrefs)}
        feats = tab_r[:, 2:2 + f]
        m_i = accm_r[0] + accm_r[1]
        hid = _silu(feats @ p['nw1f'] + m_i @ p['nw1m'] + p['nb1'])
        hid = hid @ p['nw2'] + p['nb2']
        if last:
            out_r[...] = hid
        else:
            mhat = accc_r[0, :, 0:2] + accc_r[1, :, 0:2]
            coors = tab_r[:, 0:2] + mhat
            out_r[...] = jnp.concatenate(
                [coors, hid, jnp.zeros((BN, 14), jnp.float32)], axis=1)

    full = lambda a: pl.BlockSpec(a.shape, lambda i: (0,) * a.ndim)
    in_specs = [pl.BlockSpec((BN, dt_in), lambda i: (i, 0)),
                pl.BlockSpec((NC, BN, 32), lambda i: (0, i, 0))]
    args = [tab, accm]
    if not last:
        in_specs.append(pl.BlockSpec((NC, BN, 16), lambda i: (0, i, 0)))
        args.append(accc)
    in_specs += [full(w) for w in wvals]
    args += wvals
    return pl.pallas_call(
        body, grid=(NPAD // BN,), in_specs=in_specs,
        out_specs=pl.BlockSpec((BN, dt_out), lambda i: (i, 0)),
        out_shape=jax.ShapeDtypeStruct((NPAD, dt_out), jnp.float32),
    )(*args)


def _tc_pool(feats, batch3, lwt, lb):
    nblk = N_NODES // BP

    def body(f_r, b_r, lwt_r, lb_r, out_r, acc):
        i = pl.program_id(0)

        @pl.when(i == 0)
        def _init():
            acc[...] = jnp.zeros((NUM_GRAPHS, 64), jnp.float32)

        gids = lax.broadcasted_iota(jnp.int32, (NUM_GRAPHS, 1), 0)
        onehot = (gids == b_r[0]).astype(jnp.float32)
        sums = lax.dot_general(onehot, f_r[...],
                               (((1,), (0,)), ((), ())))
        cnts = jnp.sum(onehot, axis=1, keepdims=True)
        acc[:, 0:32] += sums
        acc[:, 32:33] += cnts

        @pl.when(i == nblk - 1)
        def _fin():
            mean = acc[:, 0:32] / jnp.maximum(acc[:, 32:33], 1.0)
            out_r[...] = (jnp.sum(mean * lwt_r[...], axis=1, keepdims=True)
                          + lb_r[...])

    return pl.pallas_call(
        body, grid=(nblk,),
        in_specs=[pl.BlockSpec((BP, 32), lambda i: (i, 0)),
                  pl.BlockSpec((1, 1, BP), lambda i: (i, 0, 0)),
                  pl.BlockSpec((1, 32), lambda i: (0, 0)),
                  pl.BlockSpec((1, 1), lambda i: (0, 0))],
        out_specs=pl.BlockSpec((NUM_GRAPHS, 1), lambda i: (0, 0)),
        out_shape=jax.ShapeDtypeStruct((NUM_GRAPHS, 1), jnp.float32),
        scratch_shapes=[pltpu.VMEM((NUM_GRAPHS, 64), jnp.float32)],
    )(feats, batch3, lwt, lb)


def _prep_layer_weights(p, f):
    e_in = 2 * f + D_EDGE + 1
    w1 = p['e_w1']
    return {
        'w1i': w1[:f], 'w1j': w1[f:2 * f], 'w1e': w1[2 * f:2 * f + D_EDGE],
        'w1d': w1[2 * f + D_EDGE:2 * f + D_EDGE + 1],
        'b1': p['e_b1'][None, :],
        'w2': p['e_w2'], 'b2': p['e_b2'][None, :],
        'cw1': p['c_w1'], 'cb1': p['c_b1'][None, :],
        'cw2t': p['c_w2'].T, 'cb2': p['c_b2'][None, :],
        'cscale': p['coors_scale'][None, :],
        'nw1f': p['n_w1'][:f], 'nw1m': p['n_w1'][f:],
        'nb1': p['n_b1'][None, :], 'nw2': p['n_w2'], 'nb2': p['n_b2'][None, :],
    }


def kernel(x, edge_index, edge_attr, positions, batch, params):
    f32 = jnp.float32
    src = edge_index[0]
    dst = edge_index[1]
    pad_e = EP - N_EDGES
    src_p = jnp.pad(src, (0, pad_e)).reshape(NC, NS, JW, CHUNK)
    dst_p = jnp.pad(dst, (0, pad_e)).reshape(NC, NS, JW, CHUNK)
    ea_p = jnp.pad(edge_attr.T, ((0, 0), (0, pad_e))).astype(f32)

    gather16 = _make_sc_gather(16)
    gather48 = _make_sc_gather(48)
    scat32 = _make_sc_scatter(32)
    scat16 = _make_sc_scatter(16)

    tab = jnp.concatenate([positions.astype(f32), x.astype(f32),
                           jnp.zeros((N_NODES, 12), f32)], axis=1)
    tab = jnp.pad(tab, ((0, NPAD - N_NODES), (0, 0)))

    fdims = (2, 32, 32)
    tdims = (16, 48, 48)
    for li, name in enumerate(('conv1', 'conv2', 'conv3')):
        f = fdims[li]
        dt = tdims[li]
        wp = _prep_layer_weights(params[name], f)
        last = li == 2
        gather = gather16 if dt == 16 else gather48
        gs, gd = gather(tab, src_p, dst_p)
        m, cvals = _tc_edge(gs, gd, ea_p, wp, f, dt, not last)
        accm = scat32(m, dst_p)
        accc = None if last else scat16(cvals, dst_p)
        tab = _tc_node(tab, accm, accc, wp, f, dt, last)

    batch3 = batch.reshape(N_NODES // BP, 1, BP)
    lwt = params['lin_w'].T
    lb = params['lin_b'][None, :]
    return _tc_pool(tab, batch3, lwt, lb)


# speedup vs baseline: 3.2020x; 1.0169x over previous
import functools

import jax
import jax.numpy as jnp
from jax import lax
from jax.experimental import pallas as pl
from jax.experimental.pallas import tpu as pltpu, tpu_sc as plsc

N_NODES = 50000
N_EDGES = 800000
D_EDGE = 4
POS_DIM = 2
NUM_GRAPHS = 64

NC, NS, L = 2, 16, 16
NW = NC * NS
CHUNK = 128
EP = 819200
JW = EP // (NW * CHUNK)
NPAD = 53248
ZCH = NPAD // (NS * CHUNK)
IB = 50
GB = 8

BE = 4096
BN = 3328
BP = 2000


def _silu(v):
    return v * jax.nn.sigmoid(v)


def _make_sc_gather(dt, n):
    mesh = plsc.VectorSubcoreMesh(core_axis_name="c", subcore_axis_name="s",
                                  num_cores=NC, num_subcores=NS)
    nblk = (EP // NW) // n

    @functools.partial(
        pl.kernel, mesh=mesh,
        out_type=(jax.ShapeDtypeStruct((EP // n, n, dt), jnp.float32),
                  jax.ShapeDtypeStruct((EP // n, n, dt), jnp.float32)),
        scratch_types=[pltpu.VMEM((2, 1, n), jnp.int32),
                       pltpu.VMEM((2, 1, n), jnp.int32),
                       pltpu.VMEM((2, n, dt), jnp.float32),
                       pltpu.VMEM((2, n, dt), jnp.float32),
                       pltpu.SemaphoreType.DMA,
                       pltpu.SemaphoreType.DMA],
        compiler_params=pltpu.CompilerParams(use_tc_tiling_on_sc=False),
    )
    def k(tab_hbm, src_hbm, dst_hbm, gs_hbm, gd_hbm,
          idxs_v, idxd_v, rows_v, rowd_v, sem_s, sem_d):
        c = lax.axis_index("c")
        s = lax.axis_index("s")
        w = c * NS + s
        cbase = w * nblk

        def fire(b, buf):
            pltpu.sync_copy(src_hbm.at[c, s, b], idxs_v.at[buf])
            pltpu.sync_copy(dst_hbm.at[c, s, b], idxd_v.at[buf])
            pltpu.async_copy(tab_hbm.at[idxs_v.at[buf, 0]], rows_v.at[buf],
                             sem_s)
            pltpu.async_copy(tab_hbm.at[idxd_v.at[buf, 0]], rowd_v.at[buf],
                             sem_d)

        fire(0, 0)

        def body(b, carry):
            cb = b % 2

            @pl.when(b + 1 < nblk)
            def _fire():
                fire(b + 1, (b + 1) % 2)

            pltpu.make_async_copy(tab_hbm.at[idxs_v.at[cb, 0]], rows_v.at[cb],
                                  sem_s).wait()
            pltpu.sync_copy(rows_v.at[cb], gs_hbm.at[cbase + b])
            pltpu.make_async_copy(tab_hbm.at[idxd_v.at[cb, 0]], rowd_v.at[cb],
                                  sem_d).wait()
            pltpu.sync_copy(rowd_v.at[cb], gd_hbm.at[cbase + b])
            return carry

        lax.fori_loop(0, nblk, body, 0)

    return k


def _make_sc_scatter(dv):
    mesh = plsc.VectorSubcoreMesh(core_axis_name="c", subcore_axis_name="s",
                                  num_cores=NC, num_subcores=NS)

    @functools.partial(
        pl.kernel, mesh=mesh,
        out_type=jax.ShapeDtypeStruct((NC, NPAD, dv), jnp.float32),
        scratch_types=[pltpu.VMEM_SHARED((NPAD, dv), jnp.float32),
                       pltpu.VMEM((IB, CHUNK), jnp.int32),
                       pltpu.VMEM((2, CHUNK, dv), jnp.float32),
                       pltpu.SemaphoreType.DMA],
        compiler_params=pltpu.CompilerParams(use_tc_tiling_on_sc=False),
    )
    def k(val_hbm, dst_hbm, out_hbm, acc_sh, idx_v, val_v, sem_v):
        c = lax.axis_index("c")
        s = lax.axis_index("s")

        def zrow(r, carry):
            def zcol(kk, carry2):
                val_v[0, r, pl.ds(kk * L, L)] = jnp.zeros((L,), jnp.float32)
                return carry2
            return lax.fori_loop(0, dv // L, zcol, carry)
        lax.fori_loop(0, CHUNK, zrow, 0)

        def zacc(t, carry):
            pltpu.sync_copy(val_v.at[0],
                            acc_sh.at[pl.ds((s * ZCH + t) * CHUNK, CHUNK)])
            return carry
        lax.fori_loop(0, ZCH, zacc, 0)
        plsc.subcore_barrier()

        base = (c * NS + s) * (JW * CHUNK)

        def blk(b, carry):
            pltpu.sync_copy(dst_hbm.at[c, s, pl.ds(b * IB, IB)], idx_v)
            j0 = b * IB
            pltpu.async_copy(val_hbm.at[pl.ds(base + j0 * CHUNK, CHUNK)],
                             val_v.at[0], sem_v)

            def body(t, carry2):
                j = j0 + t

                @pl.when(t + 1 < IB)
                def _fire():
                    pltpu.async_copy(
                        val_hbm.at[pl.ds(base + (j + 1) * CHUNK, CHUNK)],
                        val_v.at[(t + 1) % 2], sem_v)

                pltpu.make_async_copy(
                    val_hbm.at[pl.ds(base + j * CHUNK, CHUNK)],
                    val_v.at[t % 2], sem_v).wait()
                pltpu.sync_copy(val_v.at[t % 2], acc_sh.at[idx_v.at[t]],
                                add=True)
                return carry2
            return lax.fori_loop(0, IB, body, carry)
        lax.fori_loop(0, JW // IB, blk, 0)
        plsc.subcore_barrier()

        def dump(t, carry):
            r0 = (s * ZCH + t) * CHUNK
            pltpu.sync_copy(acc_sh.at[pl.ds(r0, CHUNK)],
                            out_hbm.at[c, pl.ds(r0, CHUNK)])
            return carry
        lax.fori_loop(0, ZCH, dump, 0)

    return k


def _edge_block(gs, gd, ea, f, p, with_coors, pid, mout, cout):
    xj = gs[:, 2:2 + f]
    xi = gd[:, 2:2 + f]
    rel = gs[:, 0:2] - gd[:, 0:2]
    rd = jnp.sum(rel * rel, axis=1, keepdims=True)
    ea_term = lax.dot_general(ea, p['w1e'], (((0,), (0,)), ((), ())))
    h = (xi @ p['w1i'] + xj @ p['w1j'] + ea_term + rd * p['w1d']
         + p['b1'])
    h = _silu(h)
    m = _silu(h @ p['w2'] + p['b2'])
    eid = pid * BE + lax.broadcasted_iota(jnp.int32, (BE, 1), 0)
    mask = eid < N_EDGES
    mout[...] = jnp.where(mask, m, 0.0)
    if with_coors:
        ch = _silu(m @ p['cw1'] + p['cb1'])
        cw = jnp.sum(ch * p['cw2t'], axis=1, keepdims=True) + p['cb2']
        norm = jnp.sqrt(jnp.clip(rd, 1e-16))
        rel_n = rel / jnp.maximum(norm, 1e-8) * p['cscale']
        cwrel = cw * rel_n
        cpad = jnp.concatenate([cwrel, jnp.zeros((BE, 14), jnp.float32)], axis=1)
        cout[...] = jnp.where(mask, cpad, 0.0)


def _tc_edge(gs, gd, ea, wp, f, dt, n, with_coors):
    wkeys = ['w1i', 'w1j', 'w1e', 'w1d', 'b1', 'w2', 'b2']
    if with_coors:
        wkeys += ['cw1', 'cb1', 'cw2t', 'cb2', 'cscale']
    wvals = [wp[k] for k in wkeys]

    def body(gs_r, gd_r, ea_r, *rest):
        nw = len(wvals)
        wrefs = rest[:nw]
        outs = rest[nw:]
        p = {k: wr[...] for k, wr in zip(wkeys, wrefs)}
        mout = outs[0]
        cout = outs[1] if with_coors else None
        gs = gs_r[...].reshape(BE, dt)
        gd = gd_r[...].reshape(BE, dt)
        _edge_block(gs, gd, ea_r[...], f, p, with_coors,
                    pl.program_id(0), mout, cout)

    grid = EP // BE
    full = lambda a: pl.BlockSpec(a.shape, lambda i: (0,) * a.ndim)
    in_specs = [pl.BlockSpec((BE // n, n, dt), lambda i: (i, 0, 0)),
                pl.BlockSpec((BE // n, n, dt), lambda i: (i, 0, 0)),
                pl.BlockSpec((D_EDGE, BE), lambda i: (0, i))]
    in_specs += [full(w) for w in wvals]
    out_shape = [jax.ShapeDtypeStruct((EP, 32), jnp.float32)]
    out_specs = [pl.BlockSpec((BE, 32), lambda i: (i, 0))]
    if with_coors:
        out_shape.append(jax.ShapeDtypeStruct((EP, 16), jnp.float32))
        out_specs.append(pl.BlockSpec((BE, 16), lambda i: (i, 0)))
    res = pl.pallas_call(
        body, grid=(grid,), in_specs=in_specs, out_specs=out_specs,
        out_shape=out_shape,
    )(gs, gd, ea, *wvals)
    return res if with_coors else (res[0], None)


def _tc_node(tab, accm, accc, wp, f, dt_in, last):
    wkeys = ['nw1f', 'nw1m', 'nb1', 'nw2', 'nb2']
    wvals = [wp[k] for k in wkeys]
    dt_out = 32 if last else 48

    def body(tab_r, accm_r, *rest):
        if not last:
            accc_r = rest[0]
            rest = rest[1:]
        wrefs = rest[:len(wvals)]
        out_r = rest[len(wvals)]
        p = {k: wr[...] for k, wr in zip(wkeys, wrefs)}
        feats = tab_r[:, 2:2 + f]
        m_i = accm_r[0] + accm_r[1]
        hid = _silu(feats @ p['nw1f'] + m_i @ p['nw1m'] + p['nb1'])
        hid = hid @ p['nw2'] + p['nb2']
        if last:
            out_r[...] = hid
        else:
            mhat = accc_r[0, :, 0:2] + accc_r[1, :, 0:2]
            coors = tab_r[:, 0:2] + mhat
            out_r[...] = jnp.concatenate(
                [coors, hid, jnp.zeros((BN, 14), jnp.float32)], axis=1)

    full = lambda a: pl.BlockSpec(a.shape, lambda i: (0,) * a.ndim)
    in_specs = [pl.BlockSpec((BN, dt_in), lambda i: (i, 0)),
                pl.BlockSpec((NC, BN, 32), lambda i: (0, i, 0))]
    args = [tab, accm]
    if not last:
        in_specs.append(pl.BlockSpec((NC, BN, 16), lambda i: (0, i, 0)))
        args.append(accc)
    in_specs += [full(w) for w in wvals]
    args += wvals
    return pl.pallas_call(
        body, grid=(NPAD // BN,), in_specs=in_specs,
        out_specs=pl.BlockSpec((BN, dt_out), lambda i: (i, 0)),
        out_shape=jax.ShapeDtypeStruct((NPAD, dt_out), jnp.float32),
    )(*args)


def _tc_pool(feats, batch3, lwt, lb):
    nblk = N_NODES // BP

    def body(f_r, b_r, lwt_r, lb_r, out_r, acc):
        i = pl.program_id(0)

        @pl.when(i == 0)
        def _init():
            acc[...] = jnp.zeros((NUM_GRAPHS, 64), jnp.float32)

        gids = lax.broadcasted_iota(jnp.int32, (NUM_GRAPHS, 1), 0)
        onehot = (gids == b_r[0]).astype(jnp.float32)
        sums = lax.dot_general(onehot, f_r[...],
                               (((1,), (0,)), ((), ())))
        cnts = jnp.sum(onehot, axis=1, keepdims=True)
        acc[:, 0:32] += sums
        acc[:, 32:33] += cnts

        @pl.when(i == nblk - 1)
        def _fin():
            mean = acc[:, 0:32] / jnp.maximum(acc[:, 32:33], 1.0)
            out_r[...] = (jnp.sum(mean * lwt_r[...], axis=1, keepdims=True)
                          + lb_r[...])

    return pl.pallas_call(
        body, grid=(nblk,),
        in_specs=[pl.BlockSpec((BP, 32), lambda i: (i, 0)),
                  pl.BlockSpec((1, 1, BP), lambda i: (i, 0, 0)),
                  pl.BlockSpec((1, 32), lambda i: (0, 0)),
                  pl.BlockSpec((1, 1), lambda i: (0, 0))],
        out_specs=pl.BlockSpec((NUM_GRAPHS, 1), lambda i: (0, 0)),
        out_shape=jax.ShapeDtypeStruct((NUM_GRAPHS, 1), jnp.float32),
        scratch_shapes=[pltpu.VMEM((NUM_GRAPHS, 64), jnp.float32)],
    )(feats, batch3, lwt, lb)


def _prep_layer_weights(p, f):
    e_in = 2 * f + D_EDGE + 1
    w1 = p['e_w1']
    return {
        'w1i': w1[:f], 'w1j': w1[f:2 * f], 'w1e': w1[2 * f:2 * f + D_EDGE],
        'w1d': w1[2 * f + D_EDGE:2 * f + D_EDGE + 1],
        'b1': p['e_b1'][None, :],
        'w2': p['e_w2'], 'b2': p['e_b2'][None, :],
        'cw1': p['c_w1'], 'cb1': p['c_b1'][None, :],
        'cw2t': p['c_w2'].T, 'cb2': p['c_b2'][None, :],
        'cscale': p['coors_scale'][None, :],
        'nw1f': p['n_w1'][:f], 'nw1m': p['n_w1'][f:],
        'nb1': p['n_b1'][None, :], 'nw2': p['n_w2'], 'nb2': p['n_b2'][None, :],
    }


def kernel(x, edge_index, edge_attr, positions, batch, params):
    f32 = jnp.float32
    src = edge_index[0]
    dst = edge_index[1]
    pad_e = EP - N_EDGES
    src_f = jnp.pad(src, (0, pad_e))
    dst_f = jnp.pad(dst, (0, pad_e))
    dst_p = dst_f.reshape(NC, NS, JW, CHUNK)
    gidx = {n: (src_f.reshape(NC, NS, (EP // NW) // n, 1, n),
                dst_f.reshape(NC, NS, (EP // NW) // n, 1, n))
            for n in (1024, 512)}
    ea_p = jnp.pad(edge_attr.T, ((0, 0), (0, pad_e))).astype(f32)

    gather16 = _make_sc_gather(16, 1024)
    gather48 = _make_sc_gather(48, 512)
    scat32 = _make_sc_scatter(32)
    scat16 = _make_sc_scatter(16)

    tab = jnp.concatenate([positions.astype(f32), x.astype(f32),
                           jnp.zeros((N_NODES, 12), f32)], axis=1)
    tab = jnp.pad(tab, ((0, NPAD - N_NODES), (0, 0)))

    fdims = (2, 32, 32)
    tdims = (16, 48, 48)
    for li, name in enumerate(('conv1', 'conv2', 'conv3')):
        f = fdims[li]
        dt = tdims[li]
        wp = _prep_layer_weights(params[name], f)
        last = li == 2
        gather, n = (gather16, 1024) if dt == 16 else (gather48, 512)
        gs, gd = gather(tab, gidx[n][0], gidx[n][1])
        m, cvals = _tc_edge(gs, gd, ea_p, wp, f, dt, n, not last)
        accm = scat32(m, dst_p)
        accc = None if last else scat16(cvals, dst_p)
        tab = _tc_node(tab, accm, accc, wp, f, dt, last)

    batch3 = batch.reshape(N_NODES // BP, 1, BP)
    lwt = params['lin_w'].T
    lb = params['lin_b'][None, :]
    return _tc_pool(tab, batch3, lwt, lb)
